# baseline (speedup 1.0000x reference)
import functools

import jax
import jax.numpy as jnp
from jax import lax
from jax.experimental import pallas as pl
from jax.experimental.pallas import tpu as pltpu
from jax.experimental.pallas import tpu_sc as plsc

N_NODES = 10000
N_EDGES = 320000

NC = 2
NS = 16
NW = NC * NS
EPT = N_EDGES // NW
K = 40
NCHUNK = EPT // K
NB = 5
NGRP = NCHUNK // NB
RSTEP = 624
RCOPY = 640

H = 64
EPT2 = N_EDGES // NS
NCHUNK2 = EPT2 // K
NB2 = 4
NGRP2 = NCHUNK2 // NB2

@functools.lru_cache(maxsize=None)
def _mesh():
    return plsc.VectorSubcoreMesh(core_axis_name="c", subcore_axis_name="s",
                                  num_cores=NC, num_subcores=NS)


@functools.lru_cache(maxsize=None)
def _make_agg(d):
    @functools.partial(
        pl.kernel,
        out_type=jax.ShapeDtypeStruct((NC, N_NODES, d), jnp.float32),
        mesh=_mesh(),
        compiler_params=pltpu.CompilerParams(use_tc_tiling_on_sc=False),
        scratch_types=[
            pltpu.VMEM((NCHUNK, K), jnp.int32),
            pltpu.VMEM((NCHUNK, K), jnp.int32),
            pltpu.VMEM_SHARED((N_NODES, d), jnp.float32),
        ]
        + [pltpu.VMEM((K, d), jnp.float32) for _ in range(NB)]
        + [pltpu.SemaphoreType.DMA for _ in range(NB)],
    )
    def agg(g_hbm, src_hbm, dst_hbm, out_hbm, src_v, dst_v, acc,
            *rest):
        rows = rest[:NB]
        gsem = rest[NB:]
        cid = lax.axis_index("c")
        sid = lax.axis_index("s")
        wid = cid * NS + sid

        zv = jnp.zeros((16,), jnp.float32)

        def zrow(r, _):
            for c in range(d // 16):
                rows[0][r, pl.ds(c * 16, 16)] = zv
            return ()

        lax.fori_loop(0, K, zrow, (), unroll=False)

        def zcp(t, _):
            pltpu.sync_copy(rows[0], acc.at[pl.ds(sid * RSTEP + t * K, K)])
            return ()

        lax.fori_loop(0, RCOPY // K, zcp, (), unroll=False)

        pltpu.sync_copy(src_hbm.at[wid], src_v)
        pltpu.sync_copy(dst_hbm.at[wid], dst_v)
        plsc.subcore_barrier()

        for b in range(NB):
            pltpu.async_copy(g_hbm.at[src_v.at[b]], rows[b], gsem[b])

        def group(grp, _):
            base = grp * NB
            for b in range(NB):
                j = base + b
                pltpu.make_async_copy(g_hbm.at[src_v.at[j]], rows[b],
                                      gsem[b]).wait()
                pltpu.sync_copy(rows[b], acc.at[dst_v.at[j]], add=True)

                @pl.when(grp + 1 < NGRP)
                def _():
                    pltpu.async_copy(g_hbm.at[src_v.at[j + NB]],
                                     rows[b], gsem[b])
            return ()

        lax.fori_loop(0, NGRP, group, (), unroll=False)

        plsc.subcore_barrier()
        pltpu.sync_copy(acc.at[pl.ds(sid * RSTEP, RCOPY)],
                        out_hbm.at[cid, pl.ds(sid * RSTEP, RCOPY)])

    return agg


@functools.lru_cache(maxsize=None)
def _make_agg_fs():
    @functools.partial(
        pl.kernel,
        out_type=jax.ShapeDtypeStruct((NC, N_NODES, H), jnp.float32),
        mesh=_mesh(),
        compiler_params=pltpu.CompilerParams(use_tc_tiling_on_sc=False),
        scratch_types=[
            pltpu.VMEM((NCHUNK2, K), jnp.int32),
            pltpu.VMEM((NCHUNK2, K), jnp.int32),
            pltpu.VMEM_SHARED((N_NODES, H), jnp.float32),
            pltpu.VMEM_SHARED((N_NODES, H), jnp.float32),
        ]
        + [pltpu.VMEM((K, H), jnp.float32) for _ in range(NB2)]
        + [pltpu.SemaphoreType.DMA for _ in range(NB2)],
    )
    def agg_fs(gh_hbm, src_hbm, dst_hbm, out_hbm, src_v, dst_v, gsh, acc,
               *rest):
        rows = rest[:NB2]
        gsem = rest[NB2:]
        cid = lax.axis_index("c")
        sid = lax.axis_index("s")

        pltpu.sync_copy(gh_hbm.at[cid, pl.ds(sid * RSTEP, RCOPY)],
                        gsh.at[pl.ds(sid * RSTEP, RCOPY)])

        zv = jnp.zeros((16,), jnp.float32)

        def zrow(r, _):
            for c in range(H // 16):
                rows[0][r, pl.ds(c * 16, 16)] = zv
            return ()

        lax.fori_loop(0, K, zrow, (), unroll=False)

        def zcp(t, _):
            pltpu.sync_copy(rows[0], acc.at[pl.ds(sid * RSTEP + t * K, K)])
            return ()

        lax.fori_loop(0, RCOPY // K, zcp, (), unroll=False)

        pltpu.sync_copy(src_hbm.at[sid], src_v)
        pltpu.sync_copy(dst_hbm.at[sid], dst_v)
        plsc.subcore_barrier()

        for b in range(NB2):
            pltpu.async_copy(gsh.at[src_v.at[b]], rows[b], gsem[b])

        def group(grp, _):
            base = grp * NB2
            for b in range(NB2):
                j = base + b
                pltpu.make_async_copy(gsh.at[src_v.at[j]], rows[b],
                                      gsem[b]).wait()
                pltpu.sync_copy(rows[b], acc.at[dst_v.at[j]], add=True)

                @pl.when(grp + 1 < NGRP2)
                def _():
                    pltpu.async_copy(gsh.at[src_v.at[j + NB2]],
                                     rows[b], gsem[b])
            return ()

        lax.fori_loop(0, NGRP2, group, (), unroll=False)

        plsc.subcore_barrier()
        pltpu.sync_copy(acc.at[pl.ds(sid * RSTEP, RCOPY)],
                        out_hbm.at[cid, pl.ds(sid * RSTEP, RCOPY)])

    return agg_fs


DEGW = 8

@functools.lru_cache(maxsize=None)
def _make_deg():
    @functools.partial(
        pl.kernel,
        out_type=jax.ShapeDtypeStruct((NC, N_NODES, DEGW), jnp.float32),
        mesh=_mesh(),
        compiler_params=pltpu.CompilerParams(use_tc_tiling_on_sc=False),
        scratch_types=[
            pltpu.VMEM((NCHUNK, K), jnp.int32),
            pltpu.VMEM_SHARED((N_NODES, DEGW), jnp.float32),
            pltpu.VMEM((K, DEGW), jnp.float32),
            pltpu.SemaphoreType.DMA,
        ],
    )
    def deg_kernel(dst_hbm, zero_hbm, ones_hbm, out_hbm, dst_v, acc, ones_v,
                   sem):
        cid = lax.axis_index("c")
        sid = lax.axis_index("s")
        wid = cid * NS + sid

        pltpu.sync_copy(dst_hbm.at[wid], dst_v)
        pltpu.sync_copy(ones_hbm, ones_v)
        pltpu.sync_copy(zero_hbm, acc.at[pl.ds(sid * RSTEP, RCOPY)])
        plsc.subcore_barrier()

        def chunk(j, _):
            pltpu.async_copy(ones_v, acc.at[dst_v.at[j]], sem, add=True)
            return ()

        lax.fori_loop(0, NCHUNK, chunk, (), unroll=False)

        def drain(j, _):
            pltpu.make_async_copy(ones_v, acc.at[dst_v.at[j]], sem).wait()
            return ()

        lax.fori_loop(0, NCHUNK, drain, (), unroll=False)

        plsc.subcore_barrier()
        pltpu.sync_copy(acc.at[pl.ds(sid * RSTEP, RCOPY)],
                        out_hbm.at[cid, pl.ds(sid * RSTEP, RCOPY)])

    return deg_kernel


R = 2000
GRID = N_NODES // R


def _stage_a1_body(x_ref, w_ref, h_ref):
    h_ref[...] = jnp.dot(x_ref[...], w_ref[...],
                         preferred_element_type=jnp.float32)


def _stage_a1(x, w1):
    return pl.pallas_call(
        _stage_a1_body,
        grid=(GRID,),
        in_specs=[
            pl.BlockSpec((R, 128), lambda i: (i, 0)),
            pl.BlockSpec((128, 128), lambda i: (0, 0)),
        ],
        out_specs=pl.BlockSpec((R, 128), lambda i: (i, 0)),
        out_shape=jax.ShapeDtypeStruct((N_NODES, 128), jnp.float32),
    )(x, w1)


def _stage_a2_body(h_ref, degp_ref, g_ref, dis_ref):
    deg = 1.0 + degp_ref[0] + degp_ref[1]
    dis = lax.rsqrt(deg)
    g = dis * h_ref[...]
    g_ref[0] = g[:, :H]
    g_ref[1] = g[:, H:]
    dis_ref[...] = dis


def _stage_a2(h, degp):
    return pl.pallas_call(
        _stage_a2_body,
        grid=(GRID,),
        in_specs=[
            pl.BlockSpec((R, 128), lambda i: (i, 0)),
            pl.BlockSpec((2, R, 1), lambda i: (0, i, 0)),
        ],
        out_specs=[
            pl.BlockSpec((2, R, H), lambda i: (0, i, 0)),
            pl.BlockSpec((R, 1), lambda i: (i, 0)),
        ],
        out_shape=[
            jax.ShapeDtypeStruct((2, N_NODES, H), jnp.float32),
            jax.ShapeDtypeStruct((N_NODES, 1), jnp.float32),
        ],
    )(h, degp)


def _relu_matmul(aggh_ref, gprevh_ref, dis_ref, b_ref, w_ref):
    dis = dis_ref[...]
    z = dis * (aggh_ref[0] + gprevh_ref[0]) + b_ref[:, :H]
    z2 = dis * (aggh_ref[1] + gprevh_ref[1]) + b_ref[:, H:]
    hcat = jnp.concatenate([jnp.maximum(z, 0.0), jnp.maximum(z2, 0.0)],
                           axis=1)
    return dis * jnp.dot(hcat, w_ref[...],
                         preferred_element_type=jnp.float32)


def _stage_b_fs_body(aggh_ref, gprevh_ref, dis_ref, b_ref, w_ref, gnext_ref):
    g = _relu_matmul(aggh_ref, gprevh_ref, dis_ref, b_ref, w_ref)
    gnext_ref[0] = g[:, :H]
    gnext_ref[1] = g[:, H:]


def _stage_b_fs(aggh, gprevh, dis, b, w):
    return pl.pallas_call(
        _stage_b_fs_body,
        grid=(GRID,),
        in_specs=[
            pl.BlockSpec((2, R, H), lambda i: (0, i, 0)),
            pl.BlockSpec((2, R, H), lambda i: (0, i, 0)),
            pl.BlockSpec((R, 1), lambda i: (i, 0)),
            pl.BlockSpec((1, 128), lambda i: (0, 0)),
            pl.BlockSpec((128, 128), lambda i: (0, 0)),
        ],
        out_specs=pl.BlockSpec((2, R, H), lambda i: (0, i, 0)),
        out_shape=jax.ShapeDtypeStruct((2, N_NODES, H), jnp.float32),
    )(aggh, gprevh, dis, b.reshape(1, 128), w)


def _stage_b_last_body(aggh_ref, gprevh_ref, dis_ref, b_ref, w_ref,
                       gnext_ref):
    gnext_ref[...] = _relu_matmul(aggh_ref, gprevh_ref, dis_ref, b_ref,
                                  w_ref)


def _stage_b_last(aggh, gprevh, dis, b, w):
    return pl.pallas_call(
        _stage_b_last_body,
        grid=(GRID,),
        in_specs=[
            pl.BlockSpec((2, R, H), lambda i: (0, i, 0)),
            pl.BlockSpec((2, R, H), lambda i: (0, i, 0)),
            pl.BlockSpec((R, 1), lambda i: (i, 0)),
            pl.BlockSpec((1, 128), lambda i: (0, 0)),
            pl.BlockSpec((128, 64), lambda i: (0, 0)),
        ],
        out_specs=pl.BlockSpec((R, 64), lambda i: (i, 0)),
        out_shape=jax.ShapeDtypeStruct((N_NODES, 64), jnp.float32),
    )(aggh, gprevh, dis, b.reshape(1, 128), w)


def _stage_c_body(aggp_ref, g3_ref, dis_ref, b_ref, out_ref):
    out_ref[...] = (dis_ref[...] * (aggp_ref[0] + aggp_ref[1] + g3_ref[...])
                    + b_ref[...])


def _stage_c(aggp, g3, dis, b):
    d = g3.shape[1]
    return pl.pallas_call(
        _stage_c_body,
        grid=(GRID,),
        in_specs=[
            pl.BlockSpec((2, R, d), lambda i: (0, i, 0)),
            pl.BlockSpec((R, d), lambda i: (i, 0)),
            pl.BlockSpec((R, 1), lambda i: (i, 0)),
            pl.BlockSpec((1, d), lambda i: (0, 0)),
        ],
        out_specs=pl.BlockSpec((R, d), lambda i: (i, 0)),
        out_shape=jax.ShapeDtypeStruct((N_NODES, d), jnp.float32),
    )(aggp, g3, dis, b.reshape(1, d))


def kernel(x, edge_index, W1, b1, W2, b2, W3, b3):
    src = edge_index[0].astype(jnp.int32).reshape(NW, NCHUNK, K)
    dst = edge_index[1].astype(jnp.int32).reshape(NW, NCHUNK, K)
    src_t = edge_index[0].astype(jnp.int32).reshape(NS, NCHUNK2, K)
    dst_t = edge_index[1].astype(jnp.int32).reshape(NS, NCHUNK2, K)

    zero_deg = jnp.zeros((RCOPY, DEGW), jnp.float32)
    ones_deg = jnp.ones((K, DEGW), jnp.float32)

    degp = _make_deg()(dst, zero_deg, ones_deg)
    degp = degp[:, :, :1]

    agg_fs = _make_agg_fs()
    agg64 = _make_agg(64)

    h1 = _stage_a1(x, W1)
    g1h, dis = _stage_a2(h1, degp)
    agg1h = agg_fs(g1h, src_t, dst_t)
    g2h = _stage_b_fs(agg1h, g1h, dis, b1, W2)
    agg2h = agg_fs(g2h, src_t, dst_t)
    g3 = _stage_b_last(agg2h, g2h, dis, b2, W3)
    aggp3 = agg64(g3, src, dst)
    return _stage_c(aggp3, g3, dis, b3)


# speedup vs baseline: 1.4142x; 1.4142x over previous
import functools

import jax
import jax.numpy as jnp
from jax import lax
from jax.experimental import pallas as pl
from jax.experimental.pallas import tpu as pltpu
from jax.experimental.pallas import tpu_sc as plsc

N_NODES = 10000
N_EDGES = 320000

NC = 2
NS = 16
NW = NC * NS
EPT = N_EDGES // NW
K = 40
NCHUNK = EPT // K
NB = 5
NGRP = NCHUNK // NB
RSTEP = 624
RCOPY = 640

@functools.lru_cache(maxsize=None)
def _mesh():
    return plsc.VectorSubcoreMesh(core_axis_name="c", subcore_axis_name="s",
                                  num_cores=NC, num_subcores=NS)


@functools.lru_cache(maxsize=None)
def _make_agg(d, k=K, nchunk=NCHUNK, nb=NB):
    ngrp = nchunk // nb

    @functools.partial(
        pl.kernel,
        out_type=jax.ShapeDtypeStruct((NC, N_NODES, d), jnp.float32),
        mesh=_mesh(),
        compiler_params=pltpu.CompilerParams(use_tc_tiling_on_sc=False),
        scratch_types=[
            pltpu.VMEM((nchunk, k), jnp.int32),
            pltpu.VMEM((nchunk, k), jnp.int32),
            pltpu.VMEM_SHARED((N_NODES, d), jnp.float32),
        ]
        + [pltpu.VMEM((k, d), jnp.float32) for _ in range(nb)]
        + [pltpu.SemaphoreType.DMA for _ in range(nb)],
    )
    def agg(g_hbm, src_hbm, dst_hbm, out_hbm, src_v, dst_v, acc,
            *rest):
        rows = rest[:nb]
        gsem = rest[nb:]
        cid = lax.axis_index("c")
        sid = lax.axis_index("s")
        wid = cid * NS + sid

        zv = jnp.zeros((16,), jnp.float32)

        def zrow(r, _):
            for c in range(d // 16):
                rows[0][r, pl.ds(c * 16, 16)] = zv
            return ()

        lax.fori_loop(0, k, zrow, (), unroll=False)

        def zcp(t, _):
            pltpu.sync_copy(rows[0], acc.at[pl.ds(sid * RSTEP + t * k, k)])
            return ()

        lax.fori_loop(0, RCOPY // k, zcp, (), unroll=False)

        pltpu.sync_copy(src_hbm.at[wid], src_v)
        pltpu.sync_copy(dst_hbm.at[wid], dst_v)
        plsc.subcore_barrier()

        for b in range(nb):
            pltpu.async_copy(g_hbm.at[src_v.at[b]], rows[b], gsem[b])

        def group(grp, _):
            base = grp * nb
            for b in range(nb):
                j = base + b
                pltpu.make_async_copy(g_hbm.at[src_v.at[j]], rows[b],
                                      gsem[b]).wait()
                pltpu.sync_copy(rows[b], acc.at[dst_v.at[j]], add=True)

                @pl.when(grp + 1 < ngrp)
                def _():
                    pltpu.async_copy(g_hbm.at[src_v.at[j + nb]],
                                     rows[b], gsem[b])
            return ()

        lax.fori_loop(0, ngrp, group, (), unroll=False)

        plsc.subcore_barrier()
        pltpu.sync_copy(acc.at[pl.ds(sid * RSTEP, RCOPY)],
                        out_hbm.at[cid, pl.ds(sid * RSTEP, RCOPY)])

    return agg


DEGW = 16

@functools.lru_cache(maxsize=None)
def _make_deg():
    @functools.partial(
        pl.kernel,
        out_type=jax.ShapeDtypeStruct((NC, N_NODES, DEGW), jnp.float32),
        mesh=_mesh(),
        compiler_params=pltpu.CompilerParams(use_tc_tiling_on_sc=False),
        scratch_types=[
            pltpu.VMEM((NCHUNK, K), jnp.int32),
            pltpu.VMEM_SHARED((N_NODES, DEGW), jnp.float32),
            pltpu.VMEM((K, DEGW), jnp.float32),
            pltpu.SemaphoreType.DMA,
        ],
    )
    def deg_kernel(dst_hbm, zero_hbm, ones_hbm, out_hbm, dst_v, acc, ones_v,
                   sem):
        cid = lax.axis_index("c")
        sid = lax.axis_index("s")
        wid = cid * NS + sid

        pltpu.sync_copy(dst_hbm.at[wid], dst_v)
        pltpu.sync_copy(ones_hbm, ones_v)
        pltpu.sync_copy(zero_hbm, acc.at[pl.ds(sid * RSTEP, RCOPY)])
        plsc.subcore_barrier()

        def chunk(j, _):
            pltpu.async_copy(ones_v, acc.at[dst_v.at[j]], sem, add=True)
            return ()

        lax.fori_loop(0, NCHUNK, chunk, (), unroll=False)

        def drain(j, _):
            pltpu.make_async_copy(ones_v, acc.at[dst_v.at[j]], sem).wait()
            return ()

        lax.fori_loop(0, NCHUNK, drain, (), unroll=False)

        plsc.subcore_barrier()
        pltpu.sync_copy(acc.at[pl.ds(sid * RSTEP, RCOPY)],
                        out_hbm.at[cid, pl.ds(sid * RSTEP, RCOPY)])

    return deg_kernel


R = 2000
GRID = N_NODES // R


def _stage_a1_body(x_ref, w_ref, h_ref):
    h_ref[...] = jnp.dot(x_ref[...], w_ref[...],
                         preferred_element_type=jnp.float32)


def _stage_a1(x, w1):
    return pl.pallas_call(
        _stage_a1_body,
        grid=(GRID,),
        in_specs=[
            pl.BlockSpec((R, 128), lambda i: (i, 0)),
            pl.BlockSpec((128, 128), lambda i: (0, 0)),
        ],
        out_specs=pl.BlockSpec((R, 128), lambda i: (i, 0)),
        out_shape=jax.ShapeDtypeStruct((N_NODES, 128), jnp.float32),
    )(x, w1)


def _stage_a2_body(h_ref, degp_ref, g_ref, dis_ref):
    deg = 1.0 + degp_ref[0] + degp_ref[1]
    dis = lax.rsqrt(deg)
    g_ref[...] = dis * h_ref[...]
    dis_ref[...] = dis


def _stage_a2(h, degp):
    return pl.pallas_call(
        _stage_a2_body,
        grid=(GRID,),
        in_specs=[
            pl.BlockSpec((R, 128), lambda i: (i, 0)),
            pl.BlockSpec((2, R, 1), lambda i: (0, i, 0)),
        ],
        out_specs=[
            pl.BlockSpec((R, 128), lambda i: (i, 0)),
            pl.BlockSpec((R, 1), lambda i: (i, 0)),
        ],
        out_shape=[
            jax.ShapeDtypeStruct((N_NODES, 128), jnp.float32),
            jax.ShapeDtypeStruct((N_NODES, 1), jnp.float32),
        ],
    )(h, degp)


def _stage_b_body(aggp_ref, gprev_ref, dis_ref, b_ref, w_ref, gnext_ref):
    dis = dis_ref[...]
    z = dis * (aggp_ref[0] + aggp_ref[1] + gprev_ref[...]) + b_ref[...]
    h = jnp.maximum(z, 0.0)
    gnext_ref[...] = dis * jnp.dot(h, w_ref[...],
                                   preferred_element_type=jnp.float32)


def _stage_b(aggp, gprev, dis, b, w):
    d, d2 = w.shape
    return pl.pallas_call(
        _stage_b_body,
        grid=(GRID,),
        in_specs=[
            pl.BlockSpec((2, R, d), lambda i: (0, i, 0)),
            pl.BlockSpec((R, d), lambda i: (i, 0)),
            pl.BlockSpec((R, 1), lambda i: (i, 0)),
            pl.BlockSpec((1, d), lambda i: (0, 0)),
            pl.BlockSpec((d, d2), lambda i: (0, 0)),
        ],
        out_specs=pl.BlockSpec((R, d2), lambda i: (i, 0)),
        out_shape=jax.ShapeDtypeStruct((N_NODES, d2), jnp.float32),
    )(aggp, gprev, dis, b.reshape(1, d), w)


def _stage_c_body(aggp_ref, g3_ref, dis_ref, b_ref, out_ref):
    out_ref[...] = (dis_ref[...] * (aggp_ref[0] + aggp_ref[1] + g3_ref[...])
                    + b_ref[...])


def _stage_c(aggp, g3, dis, b):
    d = g3.shape[1]
    return pl.pallas_call(
        _stage_c_body,
        grid=(GRID,),
        in_specs=[
            pl.BlockSpec((2, R, d), lambda i: (0, i, 0)),
            pl.BlockSpec((R, d), lambda i: (i, 0)),
            pl.BlockSpec((R, 1), lambda i: (i, 0)),
            pl.BlockSpec((1, d), lambda i: (0, 0)),
        ],
        out_specs=pl.BlockSpec((R, d), lambda i: (i, 0)),
        out_shape=jax.ShapeDtypeStruct((N_NODES, d), jnp.float32),
    )(aggp, g3, dis, b.reshape(1, d))


def kernel(x, edge_index, W1, b1, W2, b2, W3, b3):
    src = edge_index[0].astype(jnp.int32).reshape(NW, NCHUNK, K)
    dst = edge_index[1].astype(jnp.int32).reshape(NW, NCHUNK, K)
    src80 = edge_index[0].astype(jnp.int32).reshape(NW, 125, 80)
    dst80 = edge_index[1].astype(jnp.int32).reshape(NW, 125, 80)

    zero_deg = jnp.zeros((RCOPY, DEGW), jnp.float32)
    ones_deg = jnp.ones((K, DEGW), jnp.float32)

    degp = _make_deg()(dst, zero_deg, ones_deg)
    degp = degp[:, :, :1]

    agg128 = _make_agg(128)
    agg64 = _make_agg(64, k=80, nchunk=125, nb=5)

    h1 = _stage_a1(x, W1)
    g1, dis = _stage_a2(h1, degp)
    aggp1 = agg128(g1, src, dst)
    g2 = _stage_b(aggp1, g1, dis, b1, W2)
    aggp2 = agg128(g2, src, dst)
    g3 = _stage_b(aggp2, g2, dis, b2, W3)
    aggp3 = agg64(g3, src80, dst80)
    return _stage_c(aggp3, g3, dis, b3)


# speedup vs baseline: 1.4246x; 1.0074x over previous
import functools

import jax
import jax.numpy as jnp
from jax import lax
from jax.experimental import pallas as pl
from jax.experimental.pallas import tpu as pltpu
from jax.experimental.pallas import tpu_sc as plsc

N_NODES = 10000
N_EDGES = 320000

NC = 2
NS = 16
NW = NC * NS
EPT = N_EDGES // NW
K = 40
NCHUNK = EPT // K
NB = 5
NGRP = NCHUNK // NB
RSTEP = 624
RCOPY = 640

@functools.lru_cache(maxsize=None)
def _mesh():
    return plsc.VectorSubcoreMesh(core_axis_name="c", subcore_axis_name="s",
                                  num_cores=NC, num_subcores=NS)


@functools.lru_cache(maxsize=None)
def _make_agg(d, k=K, nchunk=NCHUNK, nb=NB):
    ngrp = nchunk // nb

    @functools.partial(
        pl.kernel,
        out_type=jax.ShapeDtypeStruct((NC, N_NODES, d), jnp.float32),
        mesh=_mesh(),
        compiler_params=pltpu.CompilerParams(use_tc_tiling_on_sc=False),
        scratch_types=[
            pltpu.VMEM((nchunk, k), jnp.int32),
            pltpu.VMEM((nchunk, k), jnp.int32),
            pltpu.VMEM_SHARED((N_NODES, d), jnp.float32),
        ]
        + [pltpu.VMEM((k, d), jnp.float32) for _ in range(nb)]
        + [pltpu.SemaphoreType.DMA for _ in range(nb)],
    )
    def agg(g_hbm, src_hbm, dst_hbm, out_hbm, src_v, dst_v, acc,
            *rest):
        rows = rest[:nb]
        gsem = rest[nb:]
        cid = lax.axis_index("c")
        sid = lax.axis_index("s")
        wid = cid * NS + sid

        zv = jnp.zeros((16,), jnp.float32)

        def zrow(r, _):
            for c in range(d // 16):
                rows[0][r, pl.ds(c * 16, 16)] = zv
            return ()

        lax.fori_loop(0, k, zrow, (), unroll=False)

        def zcp(t, _):
            pltpu.sync_copy(rows[0], acc.at[pl.ds(sid * RSTEP + t * k, k)])
            return ()

        lax.fori_loop(0, RCOPY // k, zcp, (), unroll=False)

        pltpu.sync_copy(src_hbm.at[wid], src_v)
        pltpu.sync_copy(dst_hbm.at[wid], dst_v)
        plsc.subcore_barrier()

        for b in range(nb):
            pltpu.async_copy(g_hbm.at[src_v.at[b]], rows[b], gsem[b])

        def group(grp, _):
            base = grp * nb
            for b in range(nb):
                j = base + b
                pltpu.make_async_copy(g_hbm.at[src_v.at[j]], rows[b],
                                      gsem[b]).wait()
                pltpu.sync_copy(rows[b], acc.at[dst_v.at[j]], add=True)

                @pl.when(grp + 1 < ngrp)
                def _():
                    pltpu.async_copy(g_hbm.at[src_v.at[j + nb]],
                                     rows[b], gsem[b])
            return ()

        lax.fori_loop(0, ngrp, group, (), unroll=False)

        plsc.subcore_barrier()
        pltpu.sync_copy(acc.at[pl.ds(sid * RSTEP, RCOPY)],
                        out_hbm.at[cid, pl.ds(sid * RSTEP, RCOPY)])

    return agg


DEGW = 16

@functools.lru_cache(maxsize=None)
def _make_deg():
    @functools.partial(
        pl.kernel,
        out_type=jax.ShapeDtypeStruct((NC, N_NODES, DEGW), jnp.float32),
        mesh=_mesh(),
        compiler_params=pltpu.CompilerParams(use_tc_tiling_on_sc=False),
        scratch_types=[
            pltpu.VMEM((NCHUNK, K), jnp.int32),
            pltpu.VMEM_SHARED((N_NODES, DEGW), jnp.float32),
            pltpu.VMEM((K, DEGW), jnp.float32),
            pltpu.SemaphoreType.DMA,
        ],
    )
    def deg_kernel(dst_hbm, zero_hbm, ones_hbm, out_hbm, dst_v, acc, ones_v,
                   sem):
        cid = lax.axis_index("c")
        sid = lax.axis_index("s")
        wid = cid * NS + sid

        pltpu.sync_copy(dst_hbm.at[wid], dst_v)
        pltpu.sync_copy(ones_hbm, ones_v)
        pltpu.sync_copy(zero_hbm, acc.at[pl.ds(sid * RSTEP, RCOPY)])
        plsc.subcore_barrier()

        def chunk(j, _):
            pltpu.async_copy(ones_v, acc.at[dst_v.at[j]], sem, add=True)
            return ()

        lax.fori_loop(0, NCHUNK, chunk, (), unroll=False)

        def drain(j, _):
            pltpu.make_async_copy(ones_v, acc.at[dst_v.at[j]], sem).wait()
            return ()

        lax.fori_loop(0, NCHUNK, drain, (), unroll=False)

        plsc.subcore_barrier()
        pltpu.sync_copy(acc.at[pl.ds(sid * RSTEP, RCOPY)],
                        out_hbm.at[cid, pl.ds(sid * RSTEP, RCOPY)])

    return deg_kernel


R = 2000
GRID = N_NODES // R


def _stage_a_body(x_ref, w_ref, degp_ref, g_ref, dis_ref):
    deg = 1.0 + degp_ref[0] + degp_ref[1]
    dis = lax.rsqrt(deg)
    g_ref[...] = dis * jnp.dot(x_ref[...], w_ref[...],
                               preferred_element_type=jnp.float32)
    dis_ref[...] = dis


def _stage_a(x, w1, degp):
    return pl.pallas_call(
        _stage_a_body,
        grid=(GRID,),
        in_specs=[
            pl.BlockSpec((R, 128), lambda i: (i, 0)),
            pl.BlockSpec((128, 128), lambda i: (0, 0)),
            pl.BlockSpec((2, R, 1), lambda i: (0, i, 0)),
        ],
        out_specs=[
            pl.BlockSpec((R, 128), lambda i: (i, 0)),
            pl.BlockSpec((R, 1), lambda i: (i, 0)),
        ],
        out_shape=[
            jax.ShapeDtypeStruct((N_NODES, 128), jnp.float32),
            jax.ShapeDtypeStruct((N_NODES, 1), jnp.float32),
        ],
    )(x, w1, degp)


def _stage_b_body(aggp_ref, gprev_ref, dis_ref, b_ref, w_ref, gnext_ref):
    dis = dis_ref[...]
    z = dis * (aggp_ref[0] + aggp_ref[1] + gprev_ref[...]) + b_ref[...]
    h = jnp.maximum(z, 0.0)
    gnext_ref[...] = dis * jnp.dot(h, w_ref[...],
                                   preferred_element_type=jnp.float32)


def _stage_b(aggp, gprev, dis, b, w):
    d, d2 = w.shape
    return pl.pallas_call(
        _stage_b_body,
        grid=(GRID,),
        in_specs=[
            pl.BlockSpec((2, R, d), lambda i: (0, i, 0)),
            pl.BlockSpec((R, d), lambda i: (i, 0)),
            pl.BlockSpec((R, 1), lambda i: (i, 0)),
            pl.BlockSpec((1, d), lambda i: (0, 0)),
            pl.BlockSpec((d, d2), lambda i: (0, 0)),
        ],
        out_specs=pl.BlockSpec((R, d2), lambda i: (i, 0)),
        out_shape=jax.ShapeDtypeStruct((N_NODES, d2), jnp.float32),
    )(aggp, gprev, dis, b.reshape(1, d), w)


def _stage_c_body(aggp_ref, g3_ref, dis_ref, b_ref, out_ref):
    out_ref[...] = (dis_ref[...] * (aggp_ref[0] + aggp_ref[1] + g3_ref[...])
                    + b_ref[...])


def _stage_c(aggp, g3, dis, b):
    d = g3.shape[1]
    return pl.pallas_call(
        _stage_c_body,
        grid=(GRID,),
        in_specs=[
            pl.BlockSpec((2, R, d), lambda i: (0, i, 0)),
            pl.BlockSpec((R, d), lambda i: (i, 0)),
            pl.BlockSpec((R, 1), lambda i: (i, 0)),
            pl.BlockSpec((1, d), lambda i: (0, 0)),
        ],
        out_specs=pl.BlockSpec((R, d), lambda i: (i, 0)),
        out_shape=jax.ShapeDtypeStruct((N_NODES, d), jnp.float32),
    )(aggp, g3, dis, b.reshape(1, d))


def kernel(x, edge_index, W1, b1, W2, b2, W3, b3):
    src = edge_index[0].astype(jnp.int32).reshape(NW, NCHUNK, K)
    dst = edge_index[1].astype(jnp.int32).reshape(NW, NCHUNK, K)
    src80 = edge_index[0].astype(jnp.int32).reshape(NW, 125, 80)
    dst80 = edge_index[1].astype(jnp.int32).reshape(NW, 125, 80)

    zero_deg = jnp.zeros((RCOPY, DEGW), jnp.float32)
    ones_deg = jnp.ones((K, DEGW), jnp.float32)

    degp = _make_deg()(dst, zero_deg, ones_deg)
    degp = degp[:, :, :1]

    agg128 = _make_agg(128)
    agg64 = _make_agg(64, k=80, nchunk=125, nb=5)

    g1, dis = _stage_a(x, W1, degp)
    aggp1 = agg128(g1, src, dst)
    g2 = _stage_b(aggp1, g1, dis, b1, W2)
    aggp2 = agg128(g2, src, dst)
    g3 = _stage_b(aggp2, g2, dis, b2, W3)
    aggp3 = agg64(g3, src80, dst80)
    return _stage_c(aggp3, g3, dis, b3)
